# baseline (speedup 1.0000x reference)
import functools

import jax
import jax.numpy as jnp
from jax import lax
from jax.experimental import pallas as pl

_N = 8192
_K = 8192
_D = 32
_TILE = 256


def _nn_kernel(z_ref, w_ref, idx_ref, zq_ref):
    zt = z_ref[...]
    w = w_ref[...]
    dot = lax.dot_general(zt, w, (((1,), (1,)), ((), ())),
                          preferred_element_type=jnp.float32)
    zn = jnp.sum(zt * zt, axis=1, keepdims=True)
    wn = jnp.sum(w * w, axis=1)[None, :]
    sim = -(zn + wn - 2.0 * dot)
    mx = jnp.max(sim, axis=1, keepdims=True)
    kiota = lax.broadcasted_iota(jnp.int32, (_TILE, _K), 1)
    idx = jnp.min(jnp.where(sim == mx, kiota, _K), axis=1)
    idx_ref[...] = idx[:, None]
    onehot = (kiota == idx[:, None]).astype(jnp.float32)
    zq_ref[...] = lax.dot_general(onehot, w, (((1,), (0,)), ((), ())),
                                  preferred_element_type=jnp.float32)


@jax.jit
def kernel(z, W):
    shape = z.shape
    zf = z.reshape(_N, _D)
    idx, zq = pl.pallas_call(
        _nn_kernel,
        grid=(_N // _TILE,),
        in_specs=[
            pl.BlockSpec((_TILE, _D), lambda i: (i, 0)),
            pl.BlockSpec((_K, _D), lambda i: (0, 0)),
        ],
        out_specs=[
            pl.BlockSpec((_TILE, 1), lambda i: (i, 0)),
            pl.BlockSpec((_TILE, _D), lambda i: (i, 0)),
        ],
        out_shape=[
            jax.ShapeDtypeStruct((_N, 1), jnp.int32),
            jax.ShapeDtypeStruct((_N, _D), jnp.float32),
        ],
    )(zf, W)
    indices = idx.reshape(*shape[:-1], 1)
    z_q = zq.reshape(shape)
    loss = jnp.zeros((1,), dtype=z.dtype)
    return (z_q, indices, loss)


# speedup vs baseline: 1.6496x; 1.6496x over previous
import functools

import jax
import jax.numpy as jnp
from jax import lax
from jax.experimental import pallas as pl
from jax.experimental.pallas import tpu as pltpu
from jax.experimental.pallas import tpu_sc as plsc

_N = 8192
_K = 8192
_D = 32
_TILE = 256
_KB = 1024

_NC = 2
_NS = 16
_NW = _NC * _NS
_BPW = _N // _NW
_DP = 128


def _wn_kernel(w_ref, wn_ref):
    w = w_ref[...]
    wn_ref[...] = jnp.sum(w * w, axis=1)[None, :]


def _nn_kernel(z_ref, w_ref, wn_ref, idx_ref):
    zt = z_ref[...]
    w = w_ref[...]
    wn = wn_ref[...]
    zn = jnp.sum(zt * zt, axis=1, keepdims=True)
    dot2 = lax.dot_general(zt + zt, w, (((1,), (1,)), ((), ())),
                           preferred_element_type=jnp.float32)

    def sim_blk(kb):
        return dot2[:, kb * _KB:(kb + 1) * _KB] - (zn + wn[:, kb * _KB:(kb + 1) * _KB])

    run_v = sim_blk(0)
    run_b = jnp.zeros((_TILE, _KB), jnp.int32)
    for kb in range(1, _K // _KB):
        sblk = sim_blk(kb)
        upd = sblk > run_v
        run_v = jnp.maximum(run_v, sblk)
        run_b = jnp.where(upd, kb, run_b)

    kiota = lax.broadcasted_iota(jnp.int32, (_TILE, _KB), 1)
    run_i = run_b * _KB + kiota
    m = jnp.max(run_v, axis=1, keepdims=True)
    idx = jnp.min(jnp.where(run_v == m, run_i, _K), axis=1)
    idx_ref[...] = idx[:, None]


def _gather_body(table_hbm, idx_hbm, out_hbm, idx_v, rows_v, sem):
    wid = lax.axis_index("s") * _NC + lax.axis_index("c")
    base = wid * _BPW
    pltpu.sync_copy(idx_hbm.at[pl.ds(base, _BPW)], idx_v)
    pltpu.async_copy(table_hbm.at[idx_v], rows_v, sem).wait()
    pltpu.sync_copy(rows_v, out_hbm.at[pl.ds(base, _BPW)])


@functools.lru_cache(maxsize=1)
def _make_sc_gather():
    return functools.partial(
        pl.kernel,
        mesh=plsc.VectorSubcoreMesh(core_axis_name="c", subcore_axis_name="s"),
        out_type=jax.ShapeDtypeStruct((_N, _DP), jnp.float32),
        scratch_types=[
            pltpu.VMEM((_BPW,), jnp.int32),
            pltpu.VMEM((_BPW, _DP), jnp.float32),
            pltpu.SemaphoreType.DMA,
        ],
    )(_gather_body)


@jax.jit
def kernel(z, W):
    shape = z.shape
    zf = z.reshape(_N, _D)
    wn = pl.pallas_call(
        _wn_kernel,
        in_specs=[pl.BlockSpec((_K, _D), lambda: (0, 0))],
        out_specs=pl.BlockSpec((1, _K), lambda: (0, 0)),
        out_shape=jax.ShapeDtypeStruct((1, _K), jnp.float32),
    )(W)
    idx = pl.pallas_call(
        _nn_kernel,
        grid=(_N // _TILE,),
        in_specs=[
            pl.BlockSpec((_TILE, _D), lambda i: (i, 0)),
            pl.BlockSpec((_K, _D), lambda i: (0, 0)),
            pl.BlockSpec((1, _K), lambda i: (0, 0)),
        ],
        out_specs=pl.BlockSpec((_TILE, 1), lambda i: (i, 0)),
        out_shape=jax.ShapeDtypeStruct((_N, 1), jnp.int32),
        compiler_params=pltpu.CompilerParams(
            dimension_semantics=("parallel",),
        ),
    )(zf, W, wn)
    w_pad = jnp.pad(W, ((0, 0), (0, _DP - _D)))
    zq = _make_sc_gather()(w_pad, idx.reshape(_N))
    indices = idx.reshape(*shape[:-1], 1)
    z_q = zq[:, :_D].reshape(shape)
    loss = jnp.zeros((1,), dtype=z.dtype)
    return (z_q, indices, loss)


# speedup vs baseline: 1.7612x; 1.0676x over previous
import functools

import jax
import jax.numpy as jnp
from jax import lax
from jax.experimental import pallas as pl
from jax.experimental.pallas import tpu as pltpu
from jax.experimental.pallas import tpu_sc as plsc

_N = 8192
_K = 8192
_D = 32
_TILE = 1024
_KB = 1024

_NC = 2
_NS = 16
_NW = _NC * _NS
_BPW = _N // _NW
_DP = 128


def _wn_kernel(w_ref, wn_ref):
    w = w_ref[...]
    wn_ref[...] = jnp.sum(w * w, axis=1)[None, :]


def _nn_kernel(z_ref, w_ref, wn_ref, idx_ref):
    zt = z_ref[...]
    w = w_ref[...]
    wn = wn_ref[...]
    zn = jnp.sum(zt * zt, axis=1, keepdims=True)
    dot2 = lax.dot_general(zt + zt, w, (((1,), (1,)), ((), ())),
                           preferred_element_type=jnp.float32)

    def sim_blk(kb):
        return dot2[:, kb * _KB:(kb + 1) * _KB] - (zn + wn[:, kb * _KB:(kb + 1) * _KB])

    run_v = sim_blk(0)
    run_b = jnp.zeros((_TILE, _KB), jnp.int32)
    for kb in range(1, _K // _KB):
        sblk = sim_blk(kb)
        upd = sblk > run_v
        run_v = jnp.maximum(run_v, sblk)
        run_b = jnp.where(upd, kb, run_b)

    kiota = lax.broadcasted_iota(jnp.int32, (_TILE, _KB), 1)
    run_i = run_b * _KB + kiota
    m = jnp.max(run_v, axis=1, keepdims=True)
    idx = jnp.min(jnp.where(run_v == m, run_i, _K), axis=1)
    idx_ref[...] = idx[:, None]


def _gather_body(table_hbm, idx_hbm, out_hbm, idx_v, rows_v, sem):
    wid = lax.axis_index("s") * _NC + lax.axis_index("c")
    base = wid * _BPW
    pltpu.sync_copy(idx_hbm.at[pl.ds(base, _BPW)], idx_v)
    pltpu.async_copy(table_hbm.at[idx_v], rows_v, sem).wait()
    pltpu.sync_copy(rows_v, out_hbm.at[pl.ds(base, _BPW)])


@functools.lru_cache(maxsize=1)
def _make_sc_gather():
    return functools.partial(
        pl.kernel,
        mesh=plsc.VectorSubcoreMesh(core_axis_name="c", subcore_axis_name="s"),
        out_type=jax.ShapeDtypeStruct((_N, _DP), jnp.float32),
        scratch_types=[
            pltpu.VMEM((_BPW,), jnp.int32),
            pltpu.VMEM((_BPW, _DP), jnp.float32),
            pltpu.SemaphoreType.DMA,
        ],
    )(_gather_body)


@jax.jit
def kernel(z, W):
    shape = z.shape
    zf = z.reshape(_N, _D)
    wn = pl.pallas_call(
        _wn_kernel,
        in_specs=[pl.BlockSpec((_K, _D), lambda: (0, 0))],
        out_specs=pl.BlockSpec((1, _K), lambda: (0, 0)),
        out_shape=jax.ShapeDtypeStruct((1, _K), jnp.float32),
    )(W)
    idx = pl.pallas_call(
        _nn_kernel,
        grid=(_N // _TILE,),
        in_specs=[
            pl.BlockSpec((_TILE, _D), lambda i: (i, 0)),
            pl.BlockSpec((_K, _D), lambda i: (0, 0)),
            pl.BlockSpec((1, _K), lambda i: (0, 0)),
        ],
        out_specs=pl.BlockSpec((_TILE, 1), lambda i: (i, 0)),
        out_shape=jax.ShapeDtypeStruct((_N, 1), jnp.int32),
        compiler_params=pltpu.CompilerParams(
            dimension_semantics=("parallel",),
        ),
    )(zf, W, wn)
    w_pad = jnp.pad(W, ((0, 0), (0, _DP - _D)))
    zq = _make_sc_gather()(w_pad, idx.reshape(_N))
    indices = idx.reshape(*shape[:-1], 1)
    z_q = zq[:, :_D].reshape(shape)
    loss = jnp.zeros((1,), dtype=z.dtype)
    return (z_q, indices, loss)
